# baseline (speedup 1.0000x reference)
import functools

import jax
import jax.numpy as jnp
from jax.experimental import pallas as pl
from jax.experimental.pallas import tpu as pltpu

_TB = 1024


def _pad_axis(a, axis, multiple):
    pad = (-a.shape[axis]) % multiple
    if pad == 0:
        return a
    widths = [(0, 0)] * a.ndim
    widths[axis] = (0, pad)
    return jnp.pad(a, widths)


def _mlp_pipeline_kernel(n_steps, k_half, m_half, x_hbm, w1_hbm, b1_hbm,
                         w2_hbm, b2_hbm, o_hbm, x_buf, o_buf, w1_v, b1_v,
                         w2_v, b2_v, w2b_v, x_sem, o_sem, o2_sem, w_sem):
    tb = x_buf.shape[1]
    in_p = w1_v.shape[0]

    def x_in(slot, step):
        return pltpu.make_async_copy(
            x_hbm.at[pl.ds(step * tb, tb)], x_buf.at[slot], x_sem.at[slot])

    def o_out(slot, step):
        return pltpu.make_async_copy(
            o_buf.at[slot], o_hbm.at[pl.ds(step * tb, tb)], o_sem.at[slot])

    cp_w1a = pltpu.make_async_copy(
        w1_hbm.at[pl.ds(0, k_half)], w1_v.at[pl.ds(0, k_half)], w_sem.at[0])
    cp_w1b = pltpu.make_async_copy(
        w1_hbm.at[pl.ds(k_half, in_p - k_half)],
        w1_v.at[pl.ds(k_half, in_p - k_half)], w_sem.at[1])
    cp_b1 = pltpu.make_async_copy(b1_hbm, b1_v, w_sem.at[2])
    cp_w2 = pltpu.make_async_copy(w2_hbm, w2_v, w_sem.at[3])
    cp_b2 = pltpu.make_async_copy(b2_hbm, b2_v, w_sem.at[4])

    cp_w1a.start()
    x_in(0, 0).start()
    cp_w1b.start()
    cp_b1.start()
    cp_w2.start()
    cp_b2.start()

    @pl.when(n_steps > 1)
    def _():
        x_in(1, 1).start()

    def layer1(slot):
        h = jnp.dot(x_buf[slot], w1_v[...],
                    preferred_element_type=jnp.float32)
        return jnp.maximum(h + b1_v[...], 0.0).astype(jnp.bfloat16)

    def layer2(h, slot):
        out = jnp.dot(h, w2b_v[...], preferred_element_type=jnp.float32)
        o_buf[slot] = out + b2_v[...]

    cp_w1a.wait()
    x_in(0, 0).wait()
    h = jnp.dot(x_buf[0][:, :k_half], w1_v[:k_half],
                preferred_element_type=jnp.float32)
    cp_w1b.wait()
    h = h + jnp.dot(x_buf[0][:, k_half:], w1_v[k_half:],
                    preferred_element_type=jnp.float32)
    cp_b1.wait()
    h = jnp.maximum(h + b1_v[...], 0.0).astype(jnp.bfloat16)
    cp_w2.wait()
    cp_b2.wait()
    w2b_v[...] = w2_v[...].astype(jnp.bfloat16)
    layer2(h, 0)
    o_out(0, 0).start()

    if n_steps >= 3:
        def body(step, _):
            slot = jax.lax.rem(step, 2)

            @pl.when(step + 1 < n_steps)
            def _():
                x_in(slot ^ 1, step + 1).start()

            x_in(slot, step).wait()

            @pl.when(step >= 2)
            def _():
                o_out(slot, step).wait()

            layer2(layer1(slot), slot)
            o_out(slot, step).start()
            return ()

        jax.lax.fori_loop(1, n_steps - 1, body, ())

        last = n_steps - 1
        slot = last % 2
        x_in(slot, last).wait()
        h = layer1(slot)
        o_out(slot, 0).wait()
        out_a = jnp.dot(h[:m_half], w2b_v[...],
                        preferred_element_type=jnp.float32)
        o_buf[slot, :m_half] = out_a + b2_v[...]
        cp_oa = pltpu.make_async_copy(
            o_buf.at[slot].at[pl.ds(0, m_half)],
            o_hbm.at[pl.ds(last * tb, m_half)], o2_sem.at[0])
        cp_oa.start()
        out_b = jnp.dot(h[m_half:], w2b_v[...],
                        preferred_element_type=jnp.float32)
        o_buf[slot, m_half:] = out_b + b2_v[...]
        cp_ob = pltpu.make_async_copy(
            o_buf.at[slot].at[pl.ds(m_half, tb - m_half)],
            o_hbm.at[pl.ds(last * tb + m_half, tb - m_half)], o2_sem.at[1])
        cp_ob.start()

        o_out((last - 1) % 2, 0).wait()
        cp_oa.wait()
        cp_ob.wait()
    elif n_steps == 2:
        x_in(1, 1).wait()
        layer2(layer1(1), 1)
        o_out(1, 1).start()
        o_out(0, 0).wait()
        o_out(1, 0).wait()
    else:
        o_out(0, 0).wait()


def kernel(x, w1, b1, w2, b2):
    B, IN = x.shape
    OUT = w2.shape[1]

    x_p = _pad_axis(x, 1, 128)
    w1_p = _pad_axis(_pad_axis(w1, 0, 128), 1, 128)
    b1_p = _pad_axis(b1, 1, 128)
    w2_p = _pad_axis(_pad_axis(w2, 0, 128), 1, 128)
    b2_p = _pad_axis(b2, 1, 128)
    IN_P, H_P = w1_p.shape
    OUT_P = w2_p.shape[1]

    tb = _TB if B % _TB == 0 else B
    x_p = _pad_axis(x_p, 0, tb)
    n_steps = x_p.shape[0] // tb

    k_half = max(128, (IN_P // 2) // 128 * 128)
    m_half = max(8, (tb // 2) // 8 * 8)

    body = functools.partial(_mlp_pipeline_kernel, n_steps, k_half, m_half)

    out_p = pl.pallas_call(
        body,
        out_shape=jax.ShapeDtypeStruct((n_steps * tb, OUT_P), x.dtype),
        in_specs=[pl.BlockSpec(memory_space=pltpu.MemorySpace.HBM)] * 5,
        out_specs=pl.BlockSpec(memory_space=pltpu.MemorySpace.HBM),
        scratch_shapes=[
            pltpu.VMEM((2, tb, IN_P), jnp.float32),
            pltpu.VMEM((2, tb, OUT_P), jnp.float32),
            pltpu.VMEM((IN_P, H_P), jnp.float32),
            pltpu.VMEM((1, H_P), jnp.float32),
            pltpu.VMEM((H_P, OUT_P), jnp.float32),
            pltpu.VMEM((1, OUT_P), jnp.float32),
            pltpu.VMEM((H_P, OUT_P), jnp.bfloat16),
            pltpu.SemaphoreType.DMA((2,)),
            pltpu.SemaphoreType.DMA((2,)),
            pltpu.SemaphoreType.DMA((2,)),
            pltpu.SemaphoreType.DMA((5,)),
        ],
        compiler_params=pltpu.CompilerParams(
            vmem_limit_bytes=64 * 1024 * 1024,
        ),
    )(x_p, w1_p, b1_p, w2_p, b2_p)
    return out_p[:B, :OUT]


# speedup vs baseline: 1.0131x; 1.0131x over previous
import functools

import jax
import jax.numpy as jnp
from jax.experimental import pallas as pl
from jax.experimental.pallas import tpu as pltpu

_TB = 1024


def _pad_axis(a, axis, multiple):
    pad = (-a.shape[axis]) % multiple
    if pad == 0:
        return a
    widths = [(0, 0)] * a.ndim
    widths[axis] = (0, pad)
    return jnp.pad(a, widths)


def _mlp_pipeline_kernel(n_steps, m_half, x_hbm, w1_hbm, b1_hbm,
                         w2_hbm, b2_hbm, o_hbm, x_buf, o_buf, w1_v, b1_v,
                         w2_v, b2_v, w2b_v, x_sem, o_sem, o2_sem, w_sem):
    tb = x_buf.shape[1]

    def x_in(slot, step):
        return pltpu.make_async_copy(
            x_hbm.at[pl.ds(step * tb, tb)], x_buf.at[slot], x_sem.at[slot])

    def o_out(slot, step):
        return pltpu.make_async_copy(
            o_buf.at[slot], o_hbm.at[pl.ds(step * tb, tb)], o_sem.at[slot])

    cp_w1 = pltpu.make_async_copy(w1_hbm, w1_v, w_sem.at[0])
    cp_b1 = pltpu.make_async_copy(b1_hbm, b1_v, w_sem.at[1])
    cp_w2 = pltpu.make_async_copy(w2_hbm, w2_v, w_sem.at[2])
    cp_b2 = pltpu.make_async_copy(b2_hbm, b2_v, w_sem.at[3])

    cp_w1.start()
    x_in(0, 0).start()
    cp_b1.start()
    cp_w2.start()
    cp_b2.start()

    @pl.when(n_steps > 1)
    def _():
        x_in(1, 1).start()

    def layer1(slot):
        h = jnp.dot(x_buf[slot], w1_v[...],
                    preferred_element_type=jnp.float32)
        return jnp.maximum(h + b1_v[...], 0.0).astype(jnp.bfloat16)

    def layer2(h, slot):
        out = jnp.dot(h, w2b_v[...], preferred_element_type=jnp.float32)
        o_buf[slot] = out + b2_v[...]

    cp_w1.wait()
    cp_b1.wait()
    x_in(0, 0).wait()
    h = layer1(0)
    cp_w2.wait()
    cp_b2.wait()
    w2b_v[...] = w2_v[...].astype(jnp.bfloat16)
    layer2(h, 0)
    o_out(0, 0).start()

    if n_steps >= 3:
        def body(step, _):
            slot = jax.lax.rem(step, 2)

            @pl.when(step + 1 < n_steps)
            def _():
                x_in(slot ^ 1, step + 1).start()

            x_in(slot, step).wait()

            @pl.when(step >= 2)
            def _():
                o_out(slot, step).wait()

            layer2(layer1(slot), slot)
            o_out(slot, step).start()
            return ()

        jax.lax.fori_loop(1, n_steps - 1, body, ())

        last = n_steps - 1
        slot = last % 2
        x_in(slot, last).wait()
        h = layer1(slot)
        o_out(slot, 0).wait()
        out_a = jnp.dot(h[:m_half], w2b_v[...],
                        preferred_element_type=jnp.float32)
        o_buf[slot, :m_half] = out_a + b2_v[...]
        cp_oa = pltpu.make_async_copy(
            o_buf.at[slot].at[pl.ds(0, m_half)],
            o_hbm.at[pl.ds(last * tb, m_half)], o2_sem.at[0])
        cp_oa.start()
        out_b = jnp.dot(h[m_half:], w2b_v[...],
                        preferred_element_type=jnp.float32)
        o_buf[slot, m_half:] = out_b + b2_v[...]
        cp_ob = pltpu.make_async_copy(
            o_buf.at[slot].at[pl.ds(m_half, tb - m_half)],
            o_hbm.at[pl.ds(last * tb + m_half, tb - m_half)], o2_sem.at[1])
        cp_ob.start()

        o_out((last - 1) % 2, 0).wait()
        cp_oa.wait()
        cp_ob.wait()
    elif n_steps == 2:
        x_in(1, 1).wait()
        layer2(layer1(1), 1)
        o_out(1, 1).start()
        o_out(0, 0).wait()
        o_out(1, 0).wait()
    else:
        o_out(0, 0).wait()


def kernel(x, w1, b1, w2, b2):
    B, IN = x.shape
    OUT = w2.shape[1]

    x_p = _pad_axis(x, 1, 128)
    w1_p = _pad_axis(_pad_axis(w1, 0, 128), 1, 128)
    b1_p = _pad_axis(b1, 1, 128)
    w2_p = _pad_axis(_pad_axis(w2, 0, 128), 1, 128)
    b2_p = _pad_axis(b2, 1, 128)
    IN_P, H_P = w1_p.shape
    OUT_P = w2_p.shape[1]

    tb = _TB if B % _TB == 0 else B
    x_p = _pad_axis(x_p, 0, tb)
    n_steps = x_p.shape[0] // tb

    m_half = max(8, (tb // 2) // 8 * 8)

    body = functools.partial(_mlp_pipeline_kernel, n_steps, m_half)

    out_p = pl.pallas_call(
        body,
        out_shape=jax.ShapeDtypeStruct((n_steps * tb, OUT_P), x.dtype),
        in_specs=[pl.BlockSpec(memory_space=pltpu.MemorySpace.HBM)] * 5,
        out_specs=pl.BlockSpec(memory_space=pltpu.MemorySpace.HBM),
        scratch_shapes=[
            pltpu.VMEM((2, tb, IN_P), jnp.float32),
            pltpu.VMEM((2, tb, OUT_P), jnp.float32),
            pltpu.VMEM((IN_P, H_P), jnp.float32),
            pltpu.VMEM((1, H_P), jnp.float32),
            pltpu.VMEM((H_P, OUT_P), jnp.float32),
            pltpu.VMEM((1, OUT_P), jnp.float32),
            pltpu.VMEM((H_P, OUT_P), jnp.bfloat16),
            pltpu.SemaphoreType.DMA((2,)),
            pltpu.SemaphoreType.DMA((2,)),
            pltpu.SemaphoreType.DMA((2,)),
            pltpu.SemaphoreType.DMA((4,)),
        ],
        compiler_params=pltpu.CompilerParams(
            vmem_limit_bytes=64 * 1024 * 1024,
        ),
    )(x_p, w1_p, b1_p, w2_p, b2_p)
    return out_p[:B, :OUT]


# speedup vs baseline: 1.0145x; 1.0014x over previous
import functools

import jax
import jax.numpy as jnp
from jax.experimental import pallas as pl
from jax.experimental.pallas import tpu as pltpu

_TB = 1024


def _pad_axis(a, axis, multiple):
    pad = (-a.shape[axis]) % multiple
    if pad == 0:
        return a
    widths = [(0, 0)] * a.ndim
    widths[axis] = (0, pad)
    return jnp.pad(a, widths)


def _mlp_pipeline_kernel(n_steps, m_half, m0, x_hbm, w1_hbm, b1_hbm,
                         w2_hbm, b2_hbm, o_hbm, x_buf, o_buf, w1_v, b1_v,
                         w2_v, b2_v, w2b_v, x_sem, o_sem, o2_sem, x0_sem,
                         w_sem):
    tb = x_buf.shape[1]

    def x_in(slot, step):
        return pltpu.make_async_copy(
            x_hbm.at[pl.ds(step * tb, tb)], x_buf.at[slot], x_sem.at[slot])

    def o_out(slot, step):
        return pltpu.make_async_copy(
            o_buf.at[slot], o_hbm.at[pl.ds(step * tb, tb)], o_sem.at[slot])

    cp_w1 = pltpu.make_async_copy(w1_hbm, w1_v, w_sem.at[0])
    cp_b1 = pltpu.make_async_copy(b1_hbm, b1_v, w_sem.at[1])
    cp_w2 = pltpu.make_async_copy(w2_hbm, w2_v, w_sem.at[2])
    cp_b2 = pltpu.make_async_copy(b2_hbm, b2_v, w_sem.at[3])

    cp_x0a = pltpu.make_async_copy(
        x_hbm.at[pl.ds(0, m0)], x_buf.at[0].at[pl.ds(0, m0)], x0_sem.at[0])
    cp_x0b = pltpu.make_async_copy(
        x_hbm.at[pl.ds(m0, tb - m0)], x_buf.at[0].at[pl.ds(m0, tb - m0)],
        x0_sem.at[1])

    cp_w1.start()
    cp_b1.start()
    cp_x0a.start()
    cp_x0b.start()
    cp_w2.start()
    cp_b2.start()

    @pl.when(n_steps > 1)
    def _():
        x_in(1, 1).start()

    def layer1(slot):
        h = jnp.dot(x_buf[slot], w1_v[...],
                    preferred_element_type=jnp.float32)
        return jnp.maximum(h + b1_v[...], 0.0).astype(jnp.bfloat16)

    def layer2(h, slot):
        out = jnp.dot(h, w2b_v[...], preferred_element_type=jnp.float32)
        o_buf[slot] = out + b2_v[...]

    cp_w1.wait()
    cp_b1.wait()
    cp_x0a.wait()
    h_a = jnp.dot(x_buf[0][:m0], w1_v[...],
                  preferred_element_type=jnp.float32)
    h_a = jnp.maximum(h_a + b1_v[...], 0.0).astype(jnp.bfloat16)
    cp_x0b.wait()
    h_b = jnp.dot(x_buf[0][m0:], w1_v[...],
                  preferred_element_type=jnp.float32)
    h_b = jnp.maximum(h_b + b1_v[...], 0.0).astype(jnp.bfloat16)
    cp_w2.wait()
    cp_b2.wait()
    w2b_v[...] = w2_v[...].astype(jnp.bfloat16)
    out_a = jnp.dot(h_a, w2b_v[...], preferred_element_type=jnp.float32)
    o_buf[0, :m0] = out_a + b2_v[...]
    out_b = jnp.dot(h_b, w2b_v[...], preferred_element_type=jnp.float32)
    o_buf[0, m0:] = out_b + b2_v[...]
    o_out(0, 0).start()

    if n_steps >= 3:
        def one(step, slot):
            @pl.when(step + 1 < n_steps)
            def _():
                x_in(1 - slot, step + 1).start()

            x_in(slot, step).wait()

            @pl.when(step >= 2)
            def _():
                o_out(slot, step).wait()

            layer2(layer1(slot), slot)
            o_out(slot, step).start()

        if n_steps % 2 == 0:
            def pair_body(p, _):
                s1 = 2 * p - 1
                one(s1, 1)
                one(s1 + 1, 0)
                return ()

            jax.lax.fori_loop(1, (n_steps - 2) // 2 + 1, pair_body, ())
        else:
            def body(step, _):
                slot = jax.lax.rem(step, 2)
                one(step, slot)
                return ()

            jax.lax.fori_loop(1, n_steps - 1, body, ())

        last = n_steps - 1
        slot = last % 2
        x_in(slot, last).wait()
        h = layer1(slot)
        o_out(slot, 0).wait()
        out_a = jnp.dot(h[:m_half], w2b_v[...],
                        preferred_element_type=jnp.float32)
        o_buf[slot, :m_half] = out_a + b2_v[...]
        cp_oa = pltpu.make_async_copy(
            o_buf.at[slot].at[pl.ds(0, m_half)],
            o_hbm.at[pl.ds(last * tb, m_half)], o2_sem.at[0])
        cp_oa.start()
        out_b = jnp.dot(h[m_half:], w2b_v[...],
                        preferred_element_type=jnp.float32)
        o_buf[slot, m_half:] = out_b + b2_v[...]
        cp_ob = pltpu.make_async_copy(
            o_buf.at[slot].at[pl.ds(m_half, tb - m_half)],
            o_hbm.at[pl.ds(last * tb + m_half, tb - m_half)], o2_sem.at[1])
        cp_ob.start()

        o_out((last - 1) % 2, 0).wait()
        cp_oa.wait()
        cp_ob.wait()
    elif n_steps == 2:
        x_in(1, 1).wait()
        layer2(layer1(1), 1)
        o_out(1, 1).start()
        o_out(0, 0).wait()
        o_out(1, 0).wait()
    else:
        o_out(0, 0).wait()


def kernel(x, w1, b1, w2, b2):
    B, IN = x.shape
    OUT = w2.shape[1]

    x_p = _pad_axis(x, 1, 128)
    w1_p = _pad_axis(_pad_axis(w1, 0, 128), 1, 128)
    b1_p = _pad_axis(b1, 1, 128)
    w2_p = _pad_axis(_pad_axis(w2, 0, 128), 1, 128)
    b2_p = _pad_axis(b2, 1, 128)
    IN_P, H_P = w1_p.shape
    OUT_P = w2_p.shape[1]

    tb = _TB if B % _TB == 0 else B
    x_p = _pad_axis(x_p, 0, tb)
    n_steps = x_p.shape[0] // tb

    m_half = max(8, (tb // 2) // 8 * 8)
    m0 = min(256, m_half)

    body = functools.partial(_mlp_pipeline_kernel, n_steps, m_half, m0)

    out_p = pl.pallas_call(
        body,
        out_shape=jax.ShapeDtypeStruct((n_steps * tb, OUT_P), x.dtype),
        in_specs=[pl.BlockSpec(memory_space=pltpu.MemorySpace.HBM)] * 5,
        out_specs=pl.BlockSpec(memory_space=pltpu.MemorySpace.HBM),
        scratch_shapes=[
            pltpu.VMEM((2, tb, IN_P), jnp.float32),
            pltpu.VMEM((2, tb, OUT_P), jnp.float32),
            pltpu.VMEM((IN_P, H_P), jnp.float32),
            pltpu.VMEM((1, H_P), jnp.float32),
            pltpu.VMEM((H_P, OUT_P), jnp.float32),
            pltpu.VMEM((1, OUT_P), jnp.float32),
            pltpu.VMEM((H_P, OUT_P), jnp.bfloat16),
            pltpu.SemaphoreType.DMA((2,)),
            pltpu.SemaphoreType.DMA((2,)),
            pltpu.SemaphoreType.DMA((2,)),
            pltpu.SemaphoreType.DMA((2,)),
            pltpu.SemaphoreType.DMA((4,)),
        ],
        compiler_params=pltpu.CompilerParams(
            vmem_limit_bytes=64 * 1024 * 1024,
        ),
    )(x_p, w1_p, b1_p, w2_p, b2_p)
    return out_p[:B, :OUT]
